# baseline (speedup 1.0000x reference)
import functools

import jax
import jax.numpy as jnp
from jax import lax
from jax.experimental import pallas as pl
from jax.experimental.pallas import tpu as pltpu
from jax.experimental.pallas import tpu_sc as plsc

H = 128
W = 128
P = H * W
C = 256
CK = 64
NCK = C // CK
PAD = P
WS = 1.0 / P
WR = 2.0 / P
NW = 32
PPT = P // NW
NB = 72
OFFS = [(dy, dx) for dy in (-1, 0, 1) for dx in (-1, 0, 1)]

_mesh = plsc.VectorSubcoreMesh(core_axis_name="c", subcore_axis_name="s")


@functools.partial(
    pl.kernel,
    mesh=_mesh,
    compiler_params=pltpu.CompilerParams(use_tc_tiling_on_sc=False),
    out_type=[
        jax.ShapeDtypeStruct((NCK, 2, P, CK), jnp.float32),
        jax.ShapeDtypeStruct((2, P, 16), jnp.float32),
    ],
    scratch_types=[
        pltpu.VMEM_SHARED((P, CK), jnp.float32),
        pltpu.VMEM_SHARED((P, 16), jnp.float32),
        pltpu.VMEM((PPT,), jnp.int32),
        pltpu.VMEM((PPT,), jnp.int32),
        pltpu.VMEM((PPT,), jnp.int32),
        pltpu.VMEM((PPT,), jnp.int32),
        pltpu.VMEM(((NB + 2) * 128,), jnp.int32),
        pltpu.VMEM((NB, 128), jnp.int32),
        pltpu.VMEM((128,), jnp.int32),
        pltpu.VMEM((128,), jnp.int32),
        pltpu.VMEM((128, CK), jnp.float32),
        pltpu.VMEM((128, CK), jnp.float32),
        pltpu.VMEM((128, 16), jnp.float32),
        pltpu.VMEM((128, 16), jnp.float32),
        pltpu.SemaphoreType.DMA,
        pltpu.SemaphoreType.DMA,
        pltpu.SemaphoreType.DMA,
        pltpu.SemaphoreType.DMA,
    ],
)
def _sc_vote(ref8, n1y, n1x, n2y, n2x, wtab, acc_out, w_out,
             guide_sp, w_sp, n1y_v, n1x_v, n2y_v, n2x_v, gbuf, tbuf,
             wibufA, wibufB, rowsA, rowsB, wrowsA, wrowsB,
             semGA, semGB, semSA, semSB):
    cid = lax.axis_index("c")
    sid = lax.axis_index("s")
    wid = sid * 2 + cid
    base = wid * PPT

    pltpu.sync_copy(n1y.at[pl.ds(base, PPT)], n1y_v)
    pltpu.sync_copy(n1x.at[pl.ds(base, PPT)], n1x_v)
    pltpu.sync_copy(n2y.at[pl.ds(base, PPT)], n2y_v)
    pltpu.sync_copy(n2x.at[pl.ds(base, PPT)], n2x_v)

    zv16 = jnp.zeros((16,), jnp.float32)
    iot = lax.iota(jnp.int32, 16)

    for d in range(2):
        ny, nx = (n1y_v, n1x_v) if d == 0 else (n2y_v, n2x_v)
        for oi, (dy, dx) in enumerate(OFFS):
            q = d * 9 + oi

            def build(j, carry, d=d, dy=dy, dx=dx, q=q, ny=ny, nx=nx):
                p = base + j * 16 + iot
                py = lax.shift_right_logical(p, 7)
                px = lax.bitwise_and(p, W - 1)
                my = ny[pl.ds(j * 16, 16)]
                mx = nx[pl.ds(j * 16, 16)]
                if d == 0:
                    ty = py + dy
                    tx = px + dx
                    gy = my + dy
                    gx = mx + dx
                else:
                    ty = my + dy
                    tx = mx + dx
                    gy = py + dy
                    gx = px + dx
                m = ((ty >= 0) & (ty < H) & (tx >= 0) & (tx < W)
                     & (gy >= 0) & (gy < H) & (gx >= 0) & (gx < W))
                t = jnp.where(m, ty * W + tx, 0)
                g = jnp.where(m, gy * W + gx, PAD)
                gbuf[pl.ds(q * PPT + j * 16, 16)] = g
                b = q * 4 + lax.shift_right_logical(j, 3)
                col = lax.bitwise_and(j, 7) * 16
                tbuf[b, pl.ds(col, 16)] = t
                return carry

            lax.fori_loop(0, PPT // 16, build, 0)

    padv = jnp.full((16,), PAD, jnp.int32)

    def fill_pad(i, carry):
        gbuf[pl.ds(NB * 128 + i * 16, 16)] = padv
        return carry

    lax.fori_loop(0, 16, fill_pad, 0)

    def fill_rows_zero(i, carry):
        rowsA[lax.shift_right_logical(i, 2),
              pl.ds(lax.bitwise_and(i, 3) * 16, 16)] = zv16
        return carry

    def fill_wrows_zero(i, carry):
        wrowsA[i, pl.ds(0, 16)] = zv16
        return carry

    def zero_guide():
        lax.fori_loop(0, 512, fill_rows_zero, 0)

        def z(k, carry):
            pltpu.sync_copy(rowsA, guide_sp.at[pl.ds(sid * 1024 + k * 128, 128)])
            return carry

        lax.fori_loop(0, 8, z, 0)

    def zero_w():
        lax.fori_loop(0, 128, fill_wrows_zero, 0)

        def z(k, carry):
            pltpu.sync_copy(wrowsA, w_sp.at[pl.ds(sid * 1024 + k * 128, 128)])
            return carry

        lax.fori_loop(0, 8, z, 0)

    def guide_pipe(src, b_lo, n):
        def g_start(b, buf, sem):
            pltpu.async_copy(src.at[gbuf.at[pl.ds(b * 128, 128)]], buf, sem)

        def g_wait(b, buf, sem):
            pltpu.make_async_copy(
                src.at[gbuf.at[pl.ds(b * 128, 128)]], buf, sem).wait()

        def s_start(b, buf, sem):
            pltpu.async_copy(buf, guide_sp.at[tbuf.at[b]], sem, add=True)

        def s_wait(b, buf, sem):
            pltpu.make_async_copy(buf, guide_sp.at[tbuf.at[b]], sem).wait()

        g_start(b_lo, rowsA, semGA)
        g_start(b_lo + 1, rowsB, semGB)

        def it(i, carry):
            b0 = b_lo + 2 * i
            b1 = b0 + 1
            g_wait(b0, rowsA, semGA)
            s_start(b0, rowsA, semSA)
            g_wait(b1, rowsB, semGB)
            s_start(b1, rowsB, semSB)
            s_wait(b0, rowsA, semSA)
            g_start(b0 + 2, rowsA, semGA)
            s_wait(b1, rowsB, semSB)
            g_start(b1 + 2, rowsB, semGB)
            return carry

        lax.fori_loop(0, n // 2, it, 0)
        g_wait(b_lo + n, rowsA, semGA)
        g_wait(b_lo + n + 1, rowsB, semGB)

    def wib_build(wib, b):
        dbase = jnp.where(b >= NB // 2, 2, 0)

        def mk(jj, carry):
            g16 = gbuf[pl.ds(b * 128 + jj * 16, 16)]
            wib[pl.ds(jj * 16, 16)] = jnp.where(g16 == PAD, dbase + 1, dbase)
            return carry

        lax.fori_loop(0, 8, mk, 0)

    def weight_pipe():
        def g_start(wib, buf, sem):
            pltpu.async_copy(wtab.at[wib], buf, sem)

        def g_wait(wib, buf, sem):
            pltpu.make_async_copy(wtab.at[wib], buf, sem).wait()

        def s_start(b, buf, sem):
            pltpu.async_copy(buf, w_sp.at[tbuf.at[b]], sem, add=True)

        def s_wait(b, buf, sem):
            pltpu.make_async_copy(buf, w_sp.at[tbuf.at[b]], sem).wait()

        wib_build(wibufA, 0)
        g_start(wibufA, wrowsA, semGA)
        wib_build(wibufB, 1)
        g_start(wibufB, wrowsB, semGB)

        def it(i, carry):
            b0 = 2 * i
            b1 = b0 + 1
            g_wait(wibufA, wrowsA, semGA)
            s_start(b0, wrowsA, semSA)
            g_wait(wibufB, wrowsB, semGB)
            s_start(b1, wrowsB, semSB)
            wib_build(wibufA, b0 + 2)
            s_wait(b0, wrowsA, semSA)
            g_start(wibufA, wrowsA, semGA)
            wib_build(wibufB, b1 + 2)
            s_wait(b1, wrowsB, semSB)
            g_start(wibufB, wrowsB, semGB)
            return carry

        lax.fori_loop(0, NB // 2, it, 0)
        g_wait(wibufA, wrowsA, semGA)
        g_wait(wibufB, wrowsB, semGB)

    def dump_pipe(dst_row, bufX, bufY):
        hs = [None] * 8
        hg = dst_row(0, None)
        for k in range(8):
            buf = bufX if k % 2 == 0 else bufY
            hg.wait()
            hs[k] = dst_row(k, buf)
            if k < 7:
                if k >= 1:
                    hs[k - 1].wait()
                hg = dst_row(k + 1, None)
        hs[6].wait()
        hs[7].wait()

    def make_dump(src_sp, dst_hbm, bufX, bufY):
        def step(k, buf):
            off = sid * 1024 + k * 128
            if buf is None:
                b = bufX if k % 2 == 0 else bufY
                sem = semGA if k % 2 == 0 else semGB
                return pltpu.async_copy(src_sp.at[pl.ds(off, 128)], b, sem)
            sem = semSA if k % 2 == 0 else semSB
            return pltpu.async_copy(buf, dst_hbm.at[pl.ds(off, 128)], sem)
        return step

    zero_guide()
    zero_w()

    for ck in range(NCK):
        plsc.subcore_barrier()
        if ck == 0:
            weight_pipe()
        guide_pipe(ref8.at[ck], 0, NB // 2)
        guide_pipe(ref8.at[NCK + ck], NB // 2, NB // 2)
        plsc.subcore_barrier()
        dump_pipe(make_dump(guide_sp, acc_out.at[ck, cid], rowsA, rowsB),
                  rowsA, rowsB)
        if ck == 0:
            dump_pipe(make_dump(w_sp, w_out.at[cid], wrowsA, wrowsB),
                      wrowsA, wrowsB)
        if ck < NCK - 1:
            zero_guide()


def _merge_body(acc_ref, w_ref, out_ref):
    w = w_ref[0, :, 0] + w_ref[1, :, 0]
    w = jnp.where(w == 0.0, 1.0, w)
    inv = (1.0 / w)[:, None]
    for ck in range(NCK):
        g = acc_ref[ck, 0] + acc_ref[ck, 1]
        out_ref[:, ck * CK:(ck + 1) * CK] = g * inv


_merge = pl.pallas_call(
    _merge_body,
    grid=(16,),
    in_specs=[
        pl.BlockSpec((NCK, 2, 1024, CK), lambda i: (0, 0, i, 0)),
        pl.BlockSpec((2, 1024, 16), lambda i: (0, i, 0)),
    ],
    out_specs=pl.BlockSpec((1024, C), lambda i: (i, 0)),
    out_shape=jax.ShapeDtypeStruct((P, C), jnp.float32),
)


def kernel(data_A, data_BP, nnf_sr, nnf_rs, curr_layer):
    refT = data_BP[0].reshape(C, P).T
    ref_pad = jnp.concatenate(
        [refT, jnp.zeros((1, C), jnp.float32)], axis=0)
    ref4 = ref_pad.reshape(P + 1, NCK, CK).transpose(1, 0, 2)
    ref8 = jnp.concatenate([WS * ref4, WR * ref4], axis=0)
    n1y = nnf_sr[..., 0].reshape(P).astype(jnp.int32)
    n1x = nnf_sr[..., 1].reshape(P).astype(jnp.int32)
    n2y = nnf_rs[..., 0].reshape(P).astype(jnp.int32)
    n2x = nnf_rs[..., 1].reshape(P).astype(jnp.int32)
    wtab = jnp.zeros((4, 16), jnp.float32)
    wtab = wtab.at[0].set(WS).at[2].set(WR)

    acc, wparts = _sc_vote(ref8, n1y, n1x, n2y, n2x, wtab)
    guide_flat = _merge(acc, wparts)
    guide = guide_flat.T.reshape(C, H, W)
    return guide, data_A, data_BP
